# baseline (speedup 1.0000x reference)
import functools

import jax
import jax.numpy as jnp
from jax import lax
from jax.experimental import pallas as pl
from jax.experimental.pallas import tpu as pltpu
from jax.experimental.pallas import tpu_sc as plsc

H = 512
W = 512
HD = 128
D = 2 * HD
NC = 2
NS = 16
NW = NC * NS
RPW = H // NW
BR = 128
NCH = W // BR
NVEC = HD // 16

_mesh = plsc.VectorSubcoreMesh(core_axis_name="c", subcore_axis_name="s")


@functools.partial(
    pl.kernel,
    mesh=_mesh,
    out_type=jax.ShapeDtypeStruct((H, W, D), jnp.float32),
    scratch_types=[
        pltpu.VMEM((RPW, HD), jnp.float32),
        pltpu.VMEM((W, HD), jnp.float32),
        pltpu.VMEM((BR, HD), jnp.float32),
        pltpu.VMEM((BR, HD), jnp.float32),
        pltpu.SemaphoreType.DMA,
        pltpu.SemaphoreType.DMA,
        pltpu.SemaphoreType.DMA,
    ],
)
def _pe_sc(row_hbm, col_hbm, out_hbm, rows_v, col_v, blk_a, blk_b, sem_a,
           sem_b, sem_c):
    wid = lax.axis_index("s") * NC + lax.axis_index("c")
    base = wid * RPW
    col_stage = pltpu.async_copy(col_hbm, col_v, sem_c)
    pltpu.sync_copy(row_hbm.at[pl.ds(base, RPW)], rows_v)

    blks = (blk_a, blk_b)
    sems = (sem_a, sem_b)
    pending = [None, None]
    col_pending = []
    for ii in range(RPW):
        b = ii % 2
        if pending[b] is not None:
            for hnd in pending[b]:
                hnd.wait()
        blk = blks[b]
        rv = [rows_v[ii, pl.ds(v * 16, 16)] for v in range(NVEC)]

        def fill(j, _, blk=blk, rv=rv):
            for v in range(NVEC):
                blk[j, pl.ds(v * 16, 16)] = rv[v]
            return 0

        lax.fori_loop(0, BR, fill, 0)
        r = base + ii
        if ii == 0:
            col_stage.wait()
        col_pending.append(pltpu.async_copy(
            col_v, out_hbm.at[r, :, pl.ds(HD, HD)], sem_c))
        hs = []
        for c in range(NCH):
            hs.append(pltpu.async_copy(
                blk, out_hbm.at[r, pl.ds(c * BR, BR), pl.ds(0, HD)], sems[b]))
        pending[b] = hs
    for b in range(2):
        for hnd in pending[b]:
            hnd.wait()
    for hnd in col_pending:
        hnd.wait()


def kernel(row_embed, col_embed):
    return _pe_sc(row_embed, col_embed).reshape(H * W, D)


# speedup vs baseline: 1.0098x; 1.0098x over previous
import functools

import jax
import jax.numpy as jnp
from jax import lax
from jax.experimental import pallas as pl
from jax.experimental.pallas import tpu as pltpu
from jax.experimental.pallas import tpu_sc as plsc

H = 512
W = 512
HD = 128
D = 2 * HD
NC = 2
NS = 16
NW = NC * NS
RPW = H // NW
BR = 128
NCH = W // BR
NVEC = HD // 16

_mesh = plsc.VectorSubcoreMesh(core_axis_name="c", subcore_axis_name="s")


@functools.partial(
    pl.kernel,
    mesh=_mesh,
    out_type=jax.ShapeDtypeStruct((H, W, D), jnp.float32),
    scratch_types=[
        pltpu.VMEM((RPW, HD), jnp.float32),
        pltpu.VMEM((W, HD), jnp.float32),
        pltpu.VMEM((BR, HD), jnp.float32),
        pltpu.VMEM((BR, HD), jnp.float32),
        pltpu.SemaphoreType.DMA,
        pltpu.SemaphoreType.DMA,
        pltpu.SemaphoreType.DMA,
    ],
)
def _pe_sc(row_hbm, col_hbm, out_hbm, rows_v, col_v, blk_a, blk_b, sem_a,
           sem_b, sem_c):
    wid = lax.axis_index("s") * NC + lax.axis_index("c")
    base = wid * RPW
    pltpu.sync_copy(row_hbm.at[pl.ds(base, RPW)], rows_v)
    pltpu.sync_copy(col_hbm, col_v)

    blks = (blk_a, blk_b)
    sems = (sem_a, sem_b)
    pending = [None, None]
    col_pending = []
    for ii in range(RPW):
        b = ii % 2
        if pending[b] is not None:
            for hnd in pending[b]:
                hnd.wait()
        blk = blks[b]
        rv = [rows_v[ii, pl.ds(v * 16, 16)] for v in range(NVEC)]

        def fill(j, _, blk=blk, rv=rv):
            for v in range(NVEC):
                blk[j, pl.ds(v * 16, 16)] = rv[v]
            return 0

        lax.fori_loop(0, BR, fill, 0)
        r = base + ii
        col_pending.append(pltpu.async_copy(
            col_v, out_hbm.at[r, :, pl.ds(HD, HD)], sem_c))
        hs = []
        for c in range(NCH):
            hs.append(pltpu.async_copy(
                blk, out_hbm.at[r, pl.ds(c * BR, BR), pl.ds(0, HD)], sems[b]))
        pending[b] = hs
    for b in range(2):
        for hnd in pending[b]:
            hnd.wait()
    for hnd in col_pending:
        hnd.wait()


def kernel(row_embed, col_embed):
    return _pe_sc(row_embed, col_embed).reshape(H * W, D)


# speedup vs baseline: 1.0120x; 1.0021x over previous
import functools

import jax
import jax.numpy as jnp
from jax import lax
from jax.experimental import pallas as pl
from jax.experimental.pallas import tpu as pltpu
from jax.experimental.pallas import tpu_sc as plsc

H = 512
W = 512
HD = 128
D = 2 * HD
NC = 2
NS = 16
NW = NC * NS
RPW = H // NW
BR = 128
NCH = W // BR
NVEC = HD // 16

_mesh = plsc.VectorSubcoreMesh(
    core_axis_name="c", subcore_axis_name="s", num_cores=NC, num_subcores=NS)


@functools.partial(
    pl.kernel,
    mesh=_mesh,
    out_type=jax.ShapeDtypeStruct((H, W, D), jnp.float32),
    scratch_types=[
        pltpu.VMEM((RPW, HD), jnp.float32),
        pltpu.VMEM((W, HD), jnp.float32),
        pltpu.VMEM((BR, HD), jnp.float32),
        pltpu.VMEM((BR, HD), jnp.float32),
        pltpu.SemaphoreType.DMA,
        pltpu.SemaphoreType.DMA,
        pltpu.SemaphoreType.DMA,
    ],
)
def _pe_sc(row_hbm, col_hbm, out_hbm, rows_v, col_v, blk_a, blk_b, sem_a,
           sem_b, sem_c):
    wid = lax.axis_index("s") * NC + lax.axis_index("c")
    base = wid * RPW
    pltpu.sync_copy(row_hbm.at[pl.ds(base, RPW)], rows_v)
    pltpu.sync_copy(col_hbm, col_v)

    blks = (blk_a, blk_b)
    sems = (sem_a, sem_b)
    pending = [None, None]
    col_pending = []
    for ii in range(RPW):
        b = ii % 2
        if pending[b] is not None:
            for hnd in pending[b]:
                hnd.wait()
        blk = blks[b]
        rv = [rows_v[ii, pl.ds(v * 16, 16)] for v in range(NVEC)]

        def fill(j, _, blk=blk, rv=rv):
            for v in range(NVEC):
                blk[j, pl.ds(v * 16, 16)] = rv[v]
            return 0

        lax.fori_loop(0, BR, fill, 0)
        r = base + ii
        col_pending.append(pltpu.async_copy(
            col_v, out_hbm.at[r, :, pl.ds(HD, HD)], sem_c))
        hs = []
        for c in range(NCH):
            hs.append(pltpu.async_copy(
                blk, out_hbm.at[r, pl.ds(c * BR, BR), pl.ds(0, HD)], sems[b]))
        pending[b] = hs
    for b in range(2):
        for hnd in pending[b]:
            hnd.wait()
    for hnd in col_pending:
        hnd.wait()


def kernel(row_embed, col_embed):
    return _pe_sc(row_embed, col_embed).reshape(H * W, D)
